# Initial kernel scaffold; baseline (speedup 1.0000x reference)
import jax
import jax.numpy as jnp
from jax.experimental import pallas as pl


def kernel(input_values, noise, cls_token, position_embeddings):
    raise NotImplementedError("write your pallas kernel here")


# speedup vs baseline: 1.0276x; 1.0276x over previous
import jax
import jax.numpy as jnp
from jax import lax
from jax.experimental import pallas as pl
from jax.experimental.pallas import tpu as pltpu
from jax.experimental.pallas import tpu_sc as plsc

B = 32
N = 1024
D = 768
KEEP = N // 2
NC, NS, L = 2, 16, 16
NV = N // L
CH = 32
NCHUNK = KEEP // CH
DV = D // L

CNT0 = 1


def _radix_pass(shift, keys, osrc, odst, dig, hist):
    zeros = jnp.zeros((L,), jnp.int32)

    def zero_body(i, _):
        hist[pl.ds(i * L, L)] = zeros
        return 0

    lax.fori_loop(0, NV, zero_body, 0, unroll=4)

    def dig_body(j, _):
        ov = osrc[pl.ds(j * L, L)] & (N - 1)
        k = plsc.load_gather(keys, [ov])
        d = lax.shift_right_logical(k, shift) & 1023
        dig[pl.ds(j * L, L)] = d
        cnt, last = plsc.scan_count(d)
        plsc.addupdate_scatter(hist, [d], cnt - CNT0 + 1, mask=last)
        return 0

    lax.fori_loop(0, NV, dig_body, 0, unroll=2)

    def cs_body(i, carry):
        h = hist[pl.ds(i * L, L)]
        c = plsc.cumsum(h)
        hist[pl.ds(i * L, L)] = c - h + carry
        return carry + jnp.sum(h)

    lax.fori_loop(0, NV, cs_body, jnp.int32(0), unroll=4)

    def place_body(j, _):
        d = dig[pl.ds(j * L, L)]
        cnt, last = plsc.scan_count(d)
        base = plsc.load_gather(hist, [d])
        pos = (base + cnt - CNT0) & (N - 1)
        ov = osrc[pl.ds(j * L, L)]
        plsc.store_scatter(odst, [pos], ov)
        plsc.addupdate_scatter(hist, [d], cnt - CNT0 + 1, mask=last)
        return 0

    lax.fori_loop(0, NV, place_body, 0, unroll=2)


def _body(x_hbm, noise_hbm, pos_hbm, cls_hbm,
          out_hbm, idr_hbm, msk_hbm,
          keys, ord_a, ord_b, dig, hist, rank, maskb, ord_s,
          idxg0, idxg1, idxp0, idxp1, idxt,
          bufx0, bufx1, bufp0, bufp1, buft, bufq,
          sgx0, sgx1, sgp0, sgp1, so0, so1):
    c = lax.axis_index("c")
    s = lax.axis_index("s")
    b = s * NC + c
    lanes = lax.iota(jnp.int32, L)
    idxg = (idxg0, idxg1)
    idxp = (idxp0, idxp1)
    bufx = (bufx0, bufx1)
    bufp = (bufp0, bufp1)
    sgx = (sgx0, sgx1)
    sgp = (sgp0, sgp1)
    so = (so0, so1)

    pltpu.sync_copy(noise_hbm.at[pl.ds(b * N, N)], keys)

    def init_body(i, _):
        ord_a[pl.ds(i * L, L)] = lanes + i * L
        return 0

    lax.fori_loop(0, NV, init_body, 0, unroll=4)

    _radix_pass(0, keys, ord_a, ord_b, dig, hist)
    _radix_pass(10, keys, ord_b, ord_a, dig, hist)
    _radix_pass(20, keys, ord_a, ord_b, dig, hist)

    def rank_body(i, _):
        iv = ord_b[pl.ds(i * L, L)] & (N - 1)
        plsc.store_scatter(rank, [iv], lanes + i * L)
        return 0

    lax.fori_loop(0, NV, rank_body, 0, unroll=4)

    def mask_body(i, _):
        r = rank[pl.ds(i * L, L)]
        maskb[pl.ds(i * L, L)] = jnp.where(r < KEEP, 1, 0).astype(jnp.int32)
        return 0

    lax.fori_loop(0, NV, mask_body, 0, unroll=4)

    pltpu.sync_copy(rank, idr_hbm.at[pl.ds(b * N, N)])
    pltpu.sync_copy(maskb, msk_hbm.at[pl.ds(b * N, N)])

    ord_s[pl.ds(0, L)] = jnp.zeros((L,), jnp.int32)

    def shift_body(i, _):
        iv = ord_b[pl.ds(i * L, L)] & (N - 1)
        p = lanes + i * L + 1
        plsc.store_scatter(ord_s, [p & (KEEP - 1)], iv, mask=p < KEEP)
        return 0

    lax.fori_loop(0, KEEP // L, shift_body, 0, unroll=4)

    def fill_and_fire(slot, ci):
        @pl.when(ci >= 2)
        def _():
            pltpu.make_async_copy(
                bufx[slot], out_hbm.at[b, pl.ds((ci - 2) * CH, CH)], so[slot]
            ).wait()

        base = ci * CH

        def ib(j, _):
            iv = ord_s[pl.ds(base + j * L, L)] & (N - 1)
            rowid = lanes + base + j * L
            idxg[slot][pl.ds(j * L, L)] = iv + b * N
            idxp[slot][pl.ds(j * L, L)] = jnp.where(rowid == 0, 0, iv + 1)
            return 0

        lax.fori_loop(0, CH // L, ib, 0, unroll=2)
        pltpu.async_copy(x_hbm.at[idxg[slot]], bufx[slot], sgx[slot])
        pltpu.async_copy(pos_hbm.at[idxp[slot]], bufp[slot], sgp[slot])

    def process(slot, ci):
        pltpu.make_async_copy(x_hbm.at[idxg[slot]], bufx[slot], sgx[slot]).wait()
        pltpu.make_async_copy(pos_hbm.at[idxp[slot]], bufp[slot], sgp[slot]).wait()

        @pl.when(ci == 0)
        def _():
            pltpu.sync_copy(cls_hbm, bufx[slot].at[0])

        def add_row(r, _):
            def add_col(cc, _):
                sl = pl.ds(cc * L, L)
                bufx[slot][r, sl] = bufx[slot][r, sl] + bufp[slot][r, sl]
                return 0

            lax.fori_loop(0, DV, add_col, 0, unroll=4)
            return 0

        lax.fori_loop(0, CH, add_row, 0)
        pltpu.async_copy(bufx[slot], out_hbm.at[b, pl.ds(ci * CH, CH)], so[slot])

    fill_and_fire(0, jnp.int32(0))

    def outer(g, _):
        fill_and_fire(1, g * 2 + 1)
        process(0, g * 2)

        @pl.when(g * 2 + 2 < NCHUNK)
        def _():
            fill_and_fire(0, g * 2 + 2)

        process(1, g * 2 + 1)
        return 0

    lax.fori_loop(0, NCHUNK // 2, outer, 0)

    pltpu.make_async_copy(
        bufx[0], out_hbm.at[b, pl.ds((NCHUNK - 2) * CH, CH)], so[0]
    ).wait()
    pltpu.make_async_copy(
        bufx[1], out_hbm.at[b, pl.ds((NCHUNK - 1) * CH, CH)], so[1]
    ).wait()

    last_id = plsc.load_gather(ord_b, [jnp.full((L,), KEEP - 1, jnp.int32)])
    last_id = last_id & (N - 1)
    idxt[pl.ds(0, L)] = last_id + b * N
    pltpu.async_copy(x_hbm.at[idxt.at[pl.ds(0, 1)]], buft, sgx0)
    pltpu.make_async_copy(
        x_hbm.at[idxt.at[pl.ds(0, 1)]], buft, sgx0
    ).wait()
    idxt[pl.ds(0, L)] = last_id + 1
    pltpu.async_copy(pos_hbm.at[idxt.at[pl.ds(0, 1)]], bufq, sgp0)
    pltpu.make_async_copy(
        pos_hbm.at[idxt.at[pl.ds(0, 1)]], bufq, sgp0
    ).wait()

    def tail_add(cc, _):
        sl = pl.ds(cc * L, L)
        buft[0, sl] = buft[0, sl] + bufq[0, sl]
        return 0

    lax.fori_loop(0, DV, tail_add, 0, unroll=4)
    pltpu.sync_copy(buft, out_hbm.at[b, pl.ds(KEEP, 1)])


@jax.jit
def kernel(input_values, noise, cls_token, position_embeddings):
    x_flat = input_values.reshape(B * N, D)
    pos_rows = position_embeddings.reshape(1 + N, D)
    cls_vec = cls_token.reshape(D)
    noise_i32 = lax.bitcast_convert_type(noise, jnp.int32).reshape(B * N)

    mesh = plsc.VectorSubcoreMesh(
        core_axis_name="c", subcore_axis_name="s", num_cores=NC, num_subcores=NS
    )
    out, ids_restore_f, mask_f = pl.kernel(
        _body,
        out_type=[
            jax.ShapeDtypeStruct((B, KEEP + 1, D), jnp.float32),
            jax.ShapeDtypeStruct((B * N,), jnp.int32),
            jax.ShapeDtypeStruct((B * N,), jnp.int32),
        ],
        mesh=mesh,
        compiler_params=pltpu.CompilerParams(needs_layout_passes=False),
        scratch_types=[
            pltpu.VMEM((N,), jnp.int32),
            pltpu.VMEM((N,), jnp.int32),
            pltpu.VMEM((N,), jnp.int32),
            pltpu.VMEM((N,), jnp.int32),
            pltpu.VMEM((1024,), jnp.int32),
            pltpu.VMEM((N,), jnp.int32),
            pltpu.VMEM((N,), jnp.int32),
            pltpu.VMEM((KEEP,), jnp.int32),
            pltpu.VMEM((CH,), jnp.int32),
            pltpu.VMEM((CH,), jnp.int32),
            pltpu.VMEM((CH,), jnp.int32),
            pltpu.VMEM((CH,), jnp.int32),
            pltpu.VMEM((L,), jnp.int32),
            pltpu.VMEM((CH, D), jnp.float32),
            pltpu.VMEM((CH, D), jnp.float32),
            pltpu.VMEM((CH, D), jnp.float32),
            pltpu.VMEM((CH, D), jnp.float32),
            pltpu.VMEM((1, D), jnp.float32),
            pltpu.VMEM((1, D), jnp.float32),
            pltpu.SemaphoreType.DMA,
            pltpu.SemaphoreType.DMA,
            pltpu.SemaphoreType.DMA,
            pltpu.SemaphoreType.DMA,
            pltpu.SemaphoreType.DMA,
            pltpu.SemaphoreType.DMA,
        ],
    )(x_flat, noise_i32, pos_rows, cls_vec)

    mask = mask_f.reshape(B, N).astype(bool)
    ids_restore = ids_restore_f.reshape(B, N)
    return (out, mask, ids_restore)
